# Initial kernel scaffold; baseline (speedup 1.0000x reference)
import jax
import jax.numpy as jnp
from jax.experimental import pallas as pl


def kernel(lv, neighbor_idx, gamma, beta, W, b):
    raise NotImplementedError("write your pallas kernel here")


# speedup vs baseline: 2.5551x; 2.5551x over previous
import functools

import jax
import jax.numpy as jnp
from jax.experimental import pallas as pl
from jax.experimental.pallas import tpu as pltpu
from jax.experimental.pallas import tpu_sc as plsc

N = 50000
D = 128
FE = 9
NF = 128
G = 32
EPS = 1e-5

ROW_TILE = 1000
GATHER_WINDOW = 128
FLAT = FE * N
FLAT_PAD = -(-FLAT // GATHER_WINDOW) * GATHER_WINDOW


def _stats_body(lv_ref, g_ref, b_ref, scale_ref, shift_ref, acc_ref):
    i = pl.program_id(0)

    @pl.when(i == 0)
    def _():
        acc_ref[...] = jnp.zeros_like(acc_ref)

    x = lv_ref[...]
    acc_ref[0:1, :] += jnp.sum(x, axis=0, keepdims=True)
    acc_ref[1:2, :] += jnp.sum(x * x, axis=0, keepdims=True)

    @pl.when(i == pl.num_programs(0) - 1)
    def _():
        r = jax.lax.broadcasted_iota(jnp.int32, (D, D), 0) // (D // G)
        c = jax.lax.broadcasted_iota(jnp.int32, (D, D), 1) // (D // G)
        m = (r == c).astype(jnp.float32)
        gs = jnp.dot(acc_ref[0:1, :], m, preferred_element_type=jnp.float32)
        gsq = jnp.dot(acc_ref[1:2, :], m, preferred_element_type=jnp.float32)
        cnt = float((D // G) * N)
        mean = gs / cnt
        var = gsq / cnt - mean * mean
        rstd = jax.lax.rsqrt(var + EPS)
        sc = g_ref[...] * rstd
        scale_ref[...] = sc
        shift_ref[...] = b_ref[...] - mean * sc


def _norm_body(lv_ref, scale_ref, shift_ref, o_ref):
    o_ref[...] = jnp.maximum(lv_ref[...] * scale_ref[...] + shift_ref[...], 0.0)


def _mm_body(r_ref, w_ref, b_ref, o_ref):
    f = pl.program_id(1)

    @pl.when(f == 0)
    def _():
        o_ref[...] = jnp.broadcast_to(b_ref[...], o_ref.shape)

    o_ref[...] += jnp.dot(r_ref[...], w_ref[0], preferred_element_type=jnp.float32)


def _sc_gather(lv_r, idx_flat):
    total = idx_flat.shape[0]
    idx2 = idx_flat.reshape(1, total)
    mesh = plsc.VectorSubcoreMesh(core_axis_name="c", subcore_axis_name="s")

    @functools.partial(
        pl.kernel,
        out_type=jax.ShapeDtypeStruct((total, D), lv_r.dtype),
        mesh=mesh,
    )
    def gk(x_hbm, i_hbm, o_hbm):
        def body(i_vmem, o_vmem):
            pltpu.sync_copy(x_hbm.at[i_vmem.at[0]], o_vmem)

        pltpu.emit_pipeline(
            body,
            grid=(total // GATHER_WINDOW,),
            in_specs=[pl.BlockSpec((1, GATHER_WINDOW), lambda i: (0, i))],
            out_specs=[pl.BlockSpec((GATHER_WINDOW, D), lambda i: (i, 0))],
            core_axis_name=("c", "s"),
            dimension_semantics=(pltpu.PARALLEL,),
        )(i_hbm, o_hbm)

    return gk(lv_r, idx2)


def kernel(lv, neighbor_idx, gamma, beta, W, b):
    nt = N // ROW_TILE

    scale, shift = pl.pallas_call(
        _stats_body,
        grid=(nt,),
        in_specs=[
            pl.BlockSpec((ROW_TILE, D), lambda i: (i, 0)),
            pl.BlockSpec((1, D), lambda i: (0, 0)),
            pl.BlockSpec((1, D), lambda i: (0, 0)),
        ],
        out_specs=[
            pl.BlockSpec((1, D), lambda i: (0, 0)),
            pl.BlockSpec((1, D), lambda i: (0, 0)),
        ],
        out_shape=[
            jax.ShapeDtypeStruct((1, D), jnp.float32),
            jax.ShapeDtypeStruct((1, D), jnp.float32),
        ],
        scratch_shapes=[pltpu.VMEM((2, D), jnp.float32)],
    )(lv, gamma.reshape(1, D), beta.reshape(1, D))

    lv_r = pl.pallas_call(
        _norm_body,
        grid=(nt,),
        in_specs=[
            pl.BlockSpec((ROW_TILE, D), lambda i: (i, 0)),
            pl.BlockSpec((1, D), lambda i: (0, 0)),
            pl.BlockSpec((1, D), lambda i: (0, 0)),
        ],
        out_specs=pl.BlockSpec((ROW_TILE, D), lambda i: (i, 0)),
        out_shape=jax.ShapeDtypeStruct((N, D), jnp.float32),
    )(lv, scale, shift)

    idx_flat = neighbor_idx.astype(jnp.int32).T.reshape(FLAT)
    pad = jnp.arange(FLAT_PAD - FLAT, dtype=jnp.int32)
    rows = _sc_gather(lv_r, jnp.concatenate([idx_flat, pad]))

    Wr = W.reshape(FE, D, NF)
    out = pl.pallas_call(
        _mm_body,
        grid=(nt, FE),
        in_specs=[
            pl.BlockSpec((ROW_TILE, D), lambda i, f: (f * (N // ROW_TILE) + i, 0)),
            pl.BlockSpec((1, D, NF), lambda i, f: (f, 0, 0)),
            pl.BlockSpec((1, NF), lambda i, f: (0, 0)),
        ],
        out_specs=pl.BlockSpec((ROW_TILE, NF), lambda i, f: (i, 0)),
        out_shape=jax.ShapeDtypeStruct((N, NF), jnp.float32),
    )(rows, Wr, b.reshape(1, NF))

    return out


# speedup vs baseline: 4.3307x; 1.6949x over previous
import functools

import jax
import jax.numpy as jnp
from jax.experimental import pallas as pl
from jax.experimental.pallas import tpu as pltpu
from jax.experimental.pallas import tpu_sc as plsc

N = 50000
D = 128
FE = 9
NF = 128
G = 32
EPS = 1e-5

ROW_TILE = 2000
MM_TILE = 2000
GATHER_WINDOW = 128
FLAT = FE * N
FLAT_PAD = -(-FLAT // GATHER_WINDOW) * GATHER_WINDOW


def _stats_body(lv_ref, g_ref, b_ref, scale_ref, shift_ref, acc_ref):
    i = pl.program_id(0)

    @pl.when(i == 0)
    def _():
        acc_ref[...] = jnp.zeros_like(acc_ref)

    x = lv_ref[...]
    acc_ref[0:1, :] += jnp.sum(x, axis=0, keepdims=True)
    acc_ref[1:2, :] += jnp.sum(x * x, axis=0, keepdims=True)

    @pl.when(i == pl.num_programs(0) - 1)
    def _():
        r = jax.lax.broadcasted_iota(jnp.int32, (D, D), 0) // (D // G)
        c = jax.lax.broadcasted_iota(jnp.int32, (D, D), 1) // (D // G)
        m = (r == c).astype(jnp.float32)
        gs = jnp.dot(acc_ref[0:1, :], m, preferred_element_type=jnp.float32)
        gsq = jnp.dot(acc_ref[1:2, :], m, preferred_element_type=jnp.float32)
        cnt = float((D // G) * N)
        mean = gs / cnt
        var = gsq / cnt - mean * mean
        rstd = jax.lax.rsqrt(var + EPS)
        sc = g_ref[...] * rstd
        scale_ref[...] = sc
        shift_ref[...] = b_ref[...] - mean * sc


def _norm_body(lv_ref, scale_ref, shift_ref, o_ref):
    o_ref[...] = jnp.maximum(
        lv_ref[...] * scale_ref[...] + shift_ref[...], 0.0
    ).astype(o_ref.dtype)


def _mm_body(*refs):
    r_refs, (w_ref, b_ref, o_ref) = refs[:FE], refs[FE:]
    acc = jnp.broadcast_to(b_ref[...], o_ref.shape)
    for f in range(FE):
        acc = acc + jnp.dot(
            r_refs[f][...].astype(jnp.bfloat16),
            w_ref[f],
            preferred_element_type=jnp.float32,
        )
    o_ref[...] = acc


def _sc_gather(lv_r, idx_flat):
    total = idx_flat.shape[0]
    idx2 = idx_flat.reshape(1, total)
    mesh = plsc.VectorSubcoreMesh(core_axis_name="c", subcore_axis_name="s")

    @functools.partial(
        pl.kernel,
        out_type=jax.ShapeDtypeStruct((total, D), lv_r.dtype),
        mesh=mesh,
    )
    def gk(x_hbm, i_hbm, o_hbm):
        def body(i_vmem, o_vmem):
            pltpu.sync_copy(x_hbm.at[i_vmem.at[0]], o_vmem)

        pltpu.emit_pipeline(
            body,
            grid=(total // GATHER_WINDOW,),
            in_specs=[pl.BlockSpec((1, GATHER_WINDOW), lambda i: (0, i))],
            out_specs=[pl.BlockSpec((GATHER_WINDOW, D), lambda i: (i, 0))],
            core_axis_name=("c", "s"),
            dimension_semantics=(pltpu.PARALLEL,),
        )(i_hbm, o_hbm)

    return gk(lv_r, idx2)


def kernel(lv, neighbor_idx, gamma, beta, W, b):
    nt = N // ROW_TILE

    scale, shift = pl.pallas_call(
        _stats_body,
        grid=(nt,),
        in_specs=[
            pl.BlockSpec((ROW_TILE, D), lambda i: (i, 0)),
            pl.BlockSpec((1, D), lambda i: (0, 0)),
            pl.BlockSpec((1, D), lambda i: (0, 0)),
        ],
        out_specs=[
            pl.BlockSpec((1, D), lambda i: (0, 0)),
            pl.BlockSpec((1, D), lambda i: (0, 0)),
        ],
        out_shape=[
            jax.ShapeDtypeStruct((1, D), jnp.float32),
            jax.ShapeDtypeStruct((1, D), jnp.float32),
        ],
        scratch_shapes=[pltpu.VMEM((2, D), jnp.float32)],
    )(lv, gamma.reshape(1, D), beta.reshape(1, D))

    lv_r = pl.pallas_call(
        _norm_body,
        grid=(nt,),
        in_specs=[
            pl.BlockSpec((ROW_TILE, D), lambda i: (i, 0)),
            pl.BlockSpec((1, D), lambda i: (0, 0)),
            pl.BlockSpec((1, D), lambda i: (0, 0)),
        ],
        out_specs=pl.BlockSpec((ROW_TILE, D), lambda i: (i, 0)),
        out_shape=jax.ShapeDtypeStruct((N, D), jnp.float32),
    )(lv, scale, shift)

    idx_flat = neighbor_idx.astype(jnp.int32).T.reshape(FLAT)
    pad = jnp.arange(FLAT_PAD - FLAT, dtype=jnp.int32)
    rows = _sc_gather(lv_r, jnp.concatenate([idx_flat, pad]))

    Wr = W.reshape(FE, D, NF).astype(jnp.bfloat16)
    mt = N // MM_TILE
    row_specs = [
        pl.BlockSpec((MM_TILE, D), functools.partial(lambda f, i: (f * mt + i, 0), f))
        for f in range(FE)
    ]
    out = pl.pallas_call(
        _mm_body,
        grid=(mt,),
        in_specs=row_specs + [
            pl.BlockSpec((FE, D, NF), lambda i: (0, 0, 0)),
            pl.BlockSpec((1, NF), lambda i: (0, 0)),
        ],
        out_specs=pl.BlockSpec((MM_TILE, NF), lambda i: (i, 0)),
        out_shape=jax.ShapeDtypeStruct((N, NF), jnp.float32),
    )(*([rows] * FE), Wr, b.reshape(1, NF))

    return out
